# speedup vs baseline: 93.8153x; 93.8153x over previous
import jax
import jax.numpy as jnp
from jax.experimental import pallas as pl

_NUM_ACTIONS = 12
_B_BLK = 1024


def _body(x_ref, tbl_ref, o_ref):
    x = x_ref[...]
    tbl = tbl_ref[...]
    cols = []
    for a in range(_NUM_ACTIONS):
        m = (x == a).astype(jnp.float32)
        cols.append(jnp.sum(m, axis=1, keepdims=True))
    counts = jnp.concatenate(cols, axis=1)
    o_ref[...] = jax.lax.dot_general(
        counts, tbl, (((1,), (0,)), ((), ())),
        preferred_element_type=jnp.float32)


def kernel(input, action_table):
    B, A = input.shape
    D = action_table.shape[1]
    return pl.pallas_call(
        _body,
        grid=(B // _B_BLK,),
        in_specs=[
            pl.BlockSpec((_B_BLK, A), lambda i: (i, 0)),
            pl.BlockSpec((_NUM_ACTIONS, D), lambda i: (0, 0)),
        ],
        out_specs=pl.BlockSpec((_B_BLK, D), lambda i: (i, 0)),
        out_shape=jax.ShapeDtypeStruct((B, D), jnp.float32),
    )(input, action_table)
